# Initial kernel scaffold; baseline (speedup 1.0000x reference)
import jax
import jax.numpy as jnp
from jax.experimental import pallas as pl


def kernel(entity_embedding, relation_embedding, normal_embedding, heads_pos, tails_pos, rels_pos, heads_neg, tails_neg, rels_neg):
    raise NotImplementedError("write your pallas kernel here")


# speedup vs baseline: 2.2857x; 2.2857x over previous
import functools

import jax
import jax.numpy as jnp
from jax import lax
from jax.experimental import pallas as pl
from jax.experimental.pallas import tpu as pltpu
from jax.experimental.pallas import tpu_sc as plsc

N_ENTITY = 100000
N_RELATION = 1000
D = 64
B = 16384
NC = 2
NS = 16
NW = NC * NS
PER_W = B // NW
C = 128
NCHUNK = PER_W // C
NV = D // 16


def _rsqrt(x):
    x = jnp.maximum(x, 1e-12)
    i = lax.bitcast_convert_type(x, jnp.int32)
    i = jnp.int32(0x5F3759DF) - (i >> 1)
    y = lax.bitcast_convert_type(i, jnp.float32)
    for _ in range(4):
        y = y * (1.5 - 0.5 * x * y * y)
    return y


def _sc_body(ent, rel, nrm, heads_p, tails_p, rels_p, heads_n, tails_n,
             rels_n, out_p, out_n, idx_h, idx_t, idx_r,
             hbuf, tbuf, rbuf, nbuf, obuf, sem):
    wid = lax.axis_index("s") * NC + lax.axis_index("c")

    def compute_triple(i, _):
        hv = [hbuf[i, pl.ds(16 * k, 16)] for k in range(NV)]
        tv = [tbuf[i, pl.ds(16 * k, 16)] for k in range(NV)]
        rv = [rbuf[i, pl.ds(16 * k, 16)] for k in range(NV)]
        nv = [nbuf[i, pl.ds(16 * k, 16)] for k in range(NV)]
        sh = hv[0] * hv[0]
        st = tv[0] * tv[0]
        sr = rv[0] * rv[0]
        sn = nv[0] * nv[0]
        dh = hv[0] * nv[0]
        dt = tv[0] * nv[0]
        for k in range(1, NV):
            sh = sh + hv[k] * hv[k]
            st = st + tv[k] * tv[k]
            sr = sr + rv[k] * rv[k]
            sn = sn + nv[k] * nv[k]
            dh = dh + hv[k] * nv[k]
            dt = dt + tv[k] * nv[k]
        a = _rsqrt(jnp.sum(sh))
        b = _rsqrt(jnp.sum(st))
        e = _rsqrt(jnp.sum(sr))
        c = _rsqrt(jnp.sum(sn))
        g = (a * jnp.sum(dh) - b * jnp.sum(dt)) * c * c
        for k in range(NV):
            obuf[i, pl.ds(16 * k, 16)] = (
                a * hv[k] - b * tv[k] + e * rv[k] - g * nv[k])
        return _

    def process(heads, tails, rels, out):
        for j in range(NCHUNK):
            base = wid * PER_W + j * C
            pltpu.sync_copy(heads.at[pl.ds(base, C)], idx_h)
            pltpu.sync_copy(tails.at[pl.ds(base, C)], idx_t)
            pltpu.sync_copy(rels.at[pl.ds(base, C)], idx_r)
            d1 = pltpu.async_copy(ent.at[idx_h], hbuf, sem)
            d2 = pltpu.async_copy(ent.at[idx_t], tbuf, sem)
            d3 = pltpu.async_copy(rel.at[idx_r], rbuf, sem)
            d4 = pltpu.async_copy(nrm.at[idx_r], nbuf, sem)
            d1.wait()
            d2.wait()
            d3.wait()
            d4.wait()
            lax.fori_loop(0, C, compute_triple, None)
            pltpu.sync_copy(obuf, out.at[pl.ds(base, C)])

    process(heads_p, tails_p, rels_p, out_p)
    process(heads_n, tails_n, rels_n, out_n)


@functools.partial(jax.jit, donate_argnums=())
def kernel(entity_embedding, relation_embedding, normal_embedding,
           heads_pos, tails_pos, rels_pos,
           heads_neg, tails_neg, rels_neg):
    mesh = plsc.VectorSubcoreMesh(core_axis_name="c", subcore_axis_name="s")
    run = pl.kernel(
        _sc_body,
        mesh=mesh,
        compiler_params=pltpu.CompilerParams(
            needs_layout_passes=False, use_tc_tiling_on_sc=False),
        out_type=(
            jax.ShapeDtypeStruct((B, D), jnp.float32),
            jax.ShapeDtypeStruct((B, D), jnp.float32),
        ),
        scratch_types=[
            pltpu.VMEM((C,), jnp.int32),
            pltpu.VMEM((C,), jnp.int32),
            pltpu.VMEM((C,), jnp.int32),
            pltpu.VMEM((C, D), jnp.float32),
            pltpu.VMEM((C, D), jnp.float32),
            pltpu.VMEM((C, D), jnp.float32),
            pltpu.VMEM((C, D), jnp.float32),
            pltpu.VMEM((C, D), jnp.float32),
            pltpu.SemaphoreType.DMA,
        ],
    )
    return run(entity_embedding, relation_embedding, normal_embedding,
               heads_pos, tails_pos, rels_pos,
               heads_neg, tails_neg, rels_neg)
